# baseline (speedup 1.0000x reference)
import functools

import jax
import jax.numpy as jnp
import numpy as np
from jax import lax
from jax.experimental import pallas as pl
from jax.experimental.pallas import tpu as pltpu
from jax.experimental.pallas import tpu_sc as plsc

N_TOK = 4096
IN_DIM = 1024
OUT_DIM = 1024
HIDDEN = 128
GATE_DIM = 4
NE = 16
BLK = 256
CAP = 12288
NBLK = CAP // BLK
GATE_T = 512
NG = N_TOK // GATE_T
NW = 32
TPW = N_TOK // NW
LANES = 16
DCH = 16
NDC = TPW // DCH
CCH = 16
NCC = TPW // CCH
_HIMASK = -65536


def _pack_bf16_pairs(lo, hi):
    bl = lax.bitcast_convert_type(lo, jnp.int32)
    bh = lax.bitcast_convert_type(hi, jnp.int32)
    rl = lax.shift_right_logical(
        bl + 0x7FFF + (lax.shift_right_logical(bl, 16) & 1), 16)
    rh = (bh + 0x7FFF + (lax.shift_right_logical(bh, 16) & 1)) & _HIMASK
    return rl | rh


def _unpack_bf16_pairs(w):
    lo = lax.bitcast_convert_type(w << 16, jnp.float32)
    hi = lax.bitcast_convert_type(w & _HIMASK, jnp.float32)
    return lo, hi


def _gate_route_body(x_ref, wg_ref, xb_ref, w0_ref, w1_ref, d0_ref, d1_ref,
                     be_ref, e0_s, e1_s):
    i = pl.program_id(0)
    x = x_ref[...]
    xb_ref[...] = _pack_bf16_pairs(x[:, :IN_DIM // 2], x[:, IN_DIM // 2:])
    logits = jnp.maximum(
        jnp.dot(x, wg_ref[...], preferred_element_type=jnp.float32), 0.0)
    g = lax.broadcasted_iota(jnp.int32, (GATE_DIM, NE), 0)
    e = lax.broadcasted_iota(jnp.int32, (GATE_DIM, NE), 1)
    mask_t = (((e >> g) & 1) * 2 - 1).astype(jnp.float32)
    s = jnp.dot(logits, mask_t, preferred_element_type=jnp.float32)
    idx = lax.broadcasted_iota(jnp.int32, s.shape, 1)
    m1 = jnp.max(s, axis=1, keepdims=True)
    i1 = jnp.min(jnp.where(s == m1, idx, NE), axis=1, keepdims=True)
    s2 = jnp.where(idx == i1, -jnp.inf, s)
    m2 = jnp.max(s2, axis=1, keepdims=True)
    i2 = jnp.min(jnp.where(s2 == m2, idx, NE), axis=1, keepdims=True)
    r = jnp.exp(m2 - m1)
    w0_ref[...] = 1.0 / (1.0 + r)
    w1_ref[...] = r / (1.0 + r)
    e0_s[pl.ds(i * GATE_T, GATE_T), :] = i1
    e1_s[pl.ds(i * GATE_T, GATE_T), :] = i2

    @pl.when(i == NG - 1)
    def _route():
        lane = lax.broadcasted_iota(jnp.int32, (N_TOK, NE), 1)
        oh0 = (e0_s[...] == lane).astype(jnp.float32)
        oh1 = (e1_s[...] == lane).astype(jnp.float32)
        c0 = jnp.sum(oh0, axis=0, keepdims=True)
        counts = c0 + jnp.sum(oh1, axis=0, keepdims=True)
        padded = jnp.ceil(counts * (1.0 / BLK)) * BLK
        r16 = lax.broadcasted_iota(jnp.int32, (NE, NE), 0)
        cidx = lax.broadcasted_iota(jnp.int32, (NE, NE), 1)
        tri = (r16 < cidx).astype(jnp.float32)
        po = jnp.dot(padded, tri, preferred_element_type=jnp.float32)
        oh = jnp.concatenate([oh0, oh1], axis=1)
        nch = 32
        chs = N_TOK // nch
        oh3 = jnp.reshape(oh, (nch, chs, 2 * NE))
        t3 = lax.broadcasted_iota(jnp.int32, (nch, chs, chs), 1)
        u3 = lax.broadcasted_iota(jnp.int32, (nch, chs, chs), 2)
        tri3 = (u3 < t3).astype(jnp.float32)
        cum3 = lax.dot_general(
            tri3, oh3, (((2,), (1,)), ((0,), (0,))),
            preferred_element_type=jnp.float32)
        tot = jnp.sum(oh3, axis=1)
        bb = lax.broadcasted_iota(jnp.int32, (nch, nch), 0)
        cc = lax.broadcasted_iota(jnp.int32, (nch, nch), 1)
        trib = (cc < bb).astype(jnp.float32)
        pre = lax.dot_general(
            trib, tot, (((1,), (0,)), ((), ())),
            preferred_element_type=jnp.float32)
        sarr = jnp.reshape(cum3 + pre[:, None, :], (N_TOK, 2 * NE))
        r0 = sarr[:, :NE]
        r1 = sarr[:, NE:]
        rank0 = jnp.sum(r0 * oh0, axis=1, keepdims=True)
        rank1 = jnp.sum((r1 + c0) * oh1, axis=1, keepdims=True)
        base0 = jnp.sum(po * oh0, axis=1, keepdims=True)
        base1 = jnp.sum(po * oh1, axis=1, keepdims=True)
        id3 = (u3 == t3).astype(jnp.float32)
        d03 = jnp.reshape(base0 + rank0, (nch, chs, 1))
        d13 = jnp.reshape(base1 + rank1, (nch, chs, 1))
        d0_ref[...] = lax.dot_general(
            d03, id3, (((1,), (1,)), ((0,), (0,))),
            preferred_element_type=jnp.float32).astype(jnp.int32)
        d1_ref[...] = lax.dot_general(
            d13, id3, (((1,), (1,)), ((0,), (0,))),
            preferred_element_type=jnp.float32).astype(jnp.int32)
        jb = (lax.broadcasted_iota(jnp.int32, (NBLK, NE), 0) * BLK).astype(
            jnp.float32)
        en = lax.broadcasted_iota(jnp.int32, (NBLK, NE), 1)
        ind = (jb >= po) & (jb < po + padded)
        be_ref[...] = jnp.sum(jnp.where(ind, en, 0), axis=1, keepdims=True)


def _gate_route(x, wg, interpret=False):
    return pl.pallas_call(
        _gate_route_body,
        grid=(NG,),
        in_specs=[
            pl.BlockSpec((GATE_T, IN_DIM), lambda i: (i, 0)),
            pl.BlockSpec((IN_DIM, GATE_DIM), lambda i: (0, 0)),
        ],
        out_specs=[
            pl.BlockSpec((GATE_T, IN_DIM // 2), lambda i: (i, 0)),
            pl.BlockSpec((GATE_T, 1), lambda i: (i, 0)),
            pl.BlockSpec((GATE_T, 1), lambda i: (i, 0)),
            pl.BlockSpec((NW, 1, TPW), lambda i: (0, 0, 0)),
            pl.BlockSpec((NW, 1, TPW), lambda i: (0, 0, 0)),
            pl.BlockSpec((NBLK, 1), lambda i: (0, 0)),
        ],
        out_shape=[
            jax.ShapeDtypeStruct((N_TOK, IN_DIM // 2), jnp.int32),
            jax.ShapeDtypeStruct((N_TOK, 1), jnp.float32),
            jax.ShapeDtypeStruct((N_TOK, 1), jnp.float32),
            jax.ShapeDtypeStruct((NW, 1, TPW), jnp.int32),
            jax.ShapeDtypeStruct((NW, 1, TPW), jnp.int32),
            jax.ShapeDtypeStruct((NBLK, 1), jnp.int32),
        ],
        scratch_shapes=[
            pltpu.VMEM((N_TOK, 1), jnp.int32),
            pltpu.VMEM((N_TOK, 1), jnp.int32),
        ],
        interpret=interpret,
    )(x, wg)


@functools.cache
def _make_dispatch():
    mesh = plsc.VectorSubcoreMesh(core_axis_name="c", subcore_axis_name="s")

    @functools.partial(
        pl.kernel,
        out_type=jax.ShapeDtypeStruct((CAP, IN_DIM // 2), jnp.int32),
        mesh=mesh,
        scratch_types=[
            pltpu.VMEM((NDC, DCH), jnp.int32),
            pltpu.VMEM((NDC, DCH), jnp.int32),
            pltpu.VMEM((DCH, IN_DIM // 2), jnp.int32),
            pltpu.VMEM((DCH, IN_DIM // 2), jnp.int32),
            pltpu.SemaphoreType.DMA,
            pltpu.SemaphoreType.DMA,
            pltpu.SemaphoreType.DMA,
            pltpu.SemaphoreType.DMA,
        ],
    )
    def _dispatch(x_hbm, d0_hbm, d1_hbm, xs_hbm, i0_v, i1_v, rows_a, rows_b,
                  sem_ra, sem_rb, sem_sa, sem_sb):
        wid = lax.axis_index("s") * 2 + lax.axis_index("c")
        base = wid * TPW
        pltpu.sync_copy(d0_hbm.at[wid], i0_v)
        pltpu.sync_copy(d1_hbm.at[wid], i1_v)
        rows = (rows_a, rows_b)
        sem_r = (sem_ra, sem_rb)
        sem_s = (sem_sa, sem_sb)
        rd = {}
        sc = {}
        rd[0] = pltpu.async_copy(
            x_hbm.at[pl.ds(base, DCH)], rows[0], sem_r[0])
        for c in range(NDC):
            if c >= 1:
                for d in sc.pop(c - 1):
                    d.wait()
            if c + 1 < NDC:
                b = (c + 1) % 2
                rd[c + 1] = pltpu.async_copy(
                    x_hbm.at[pl.ds(base + (c + 1) * DCH, DCH)],
                    rows[b], sem_r[b])
            rd.pop(c).wait()
            b = c % 2
            sc[c] = (
                pltpu.async_copy(rows[b], xs_hbm.at[i0_v.at[c]], sem_s[b]),
                pltpu.async_copy(rows[b], xs_hbm.at[i1_v.at[c]], sem_s[b]),
            )
        for d in sc.pop(NDC - 1):
            d.wait()

    return _dispatch


def _mlp_body(be_ref, xs_ref, w1_ref, b1_ref, w2_ref, b2_ref, ys_ref):
    del be_ref
    lo, hi = _unpack_bf16_pairs(xs_ref[...])
    xb = jnp.concatenate(
        [lo.astype(jnp.bfloat16), hi.astype(jnp.bfloat16)], axis=1)
    h = jnp.maximum(
        jnp.dot(xb, w1_ref[0].astype(jnp.bfloat16),
                preferred_element_type=jnp.float32) + b1_ref[0], 0.0)
    ys = jnp.dot(h.astype(jnp.bfloat16), w2_ref[0].astype(jnp.bfloat16),
                 preferred_element_type=jnp.float32) + b2_ref[0]
    ys_ref[...] = _pack_bf16_pairs(ys[:, :OUT_DIM // 2], ys[:, OUT_DIM // 2:])


def _mlp(bexp, xs, w1b, b1, w2b, b2, interpret=False):
    grid_spec = pltpu.PrefetchScalarGridSpec(
        num_scalar_prefetch=1,
        grid=(NBLK,),
        in_specs=[
            pl.BlockSpec((BLK, IN_DIM // 2), lambda i, be: (i, 0)),
            pl.BlockSpec((1, IN_DIM, HIDDEN), lambda i, be: (be[i], 0, 0)),
            pl.BlockSpec((1, 1, HIDDEN), lambda i, be: (be[i], 0, 0)),
            pl.BlockSpec((1, HIDDEN, OUT_DIM), lambda i, be: (be[i], 0, 0)),
            pl.BlockSpec((1, 1, OUT_DIM), lambda i, be: (be[i], 0, 0)),
        ],
        out_specs=pl.BlockSpec((BLK, OUT_DIM // 2), lambda i, be: (i, 0)),
    )
    return pl.pallas_call(
        _mlp_body,
        grid_spec=grid_spec,
        out_shape=jax.ShapeDtypeStruct((CAP, OUT_DIM // 2), jnp.int32),
        interpret=interpret,
    )(bexp, xs, w1b, jnp.reshape(b1, (NE, 1, HIDDEN)),
      w2b, jnp.reshape(b2, (NE, 1, OUT_DIM)))


def _splat(vec, i):
    dnums = lax.GatherDimensionNumbers(
        offset_dims=(), collapsed_slice_dims=(0,), start_index_map=(0,))
    idx = jnp.full((LANES, 1), i, jnp.int32)
    return lax.gather(vec, idx, dnums, slice_sizes=(1,),
                      mode=lax.GatherScatterMode.PROMISE_IN_BOUNDS)


@functools.cache
def _make_combine():
    mesh = plsc.VectorSubcoreMesh(core_axis_name="c", subcore_axis_name="s")

    @functools.partial(
        pl.kernel,
        out_type=jax.ShapeDtypeStruct((N_TOK, OUT_DIM), jnp.float32),
        mesh=mesh,
        scratch_types=[
            pltpu.VMEM((NCC, CCH), jnp.int32),
            pltpu.VMEM((NCC, CCH), jnp.int32),
            pltpu.VMEM((TPW,), jnp.float32),
            pltpu.VMEM((TPW,), jnp.float32),
            pltpu.VMEM((CCH, OUT_DIM // 2), jnp.int32),
            pltpu.VMEM((CCH, OUT_DIM // 2), jnp.int32),
            pltpu.VMEM((CCH, OUT_DIM // 2), jnp.int32),
            pltpu.VMEM((CCH, OUT_DIM // 2), jnp.int32),
            pltpu.VMEM((CCH, OUT_DIM), jnp.float32),
            pltpu.VMEM((CCH, OUT_DIM), jnp.float32),
            pltpu.SemaphoreType.DMA,
            pltpu.SemaphoreType.DMA,
            pltpu.SemaphoreType.DMA,
            pltpu.SemaphoreType.DMA,
        ],
    )
    def _combine(ys_hbm, d0_hbm, d1_hbm, w0_hbm, w1_hbm, out_hbm,
                 i0_v, i1_v, w0_v, w1_v, r0_a, r1_a, r0_b, r1_b, o_a, o_b,
                 sem_ga, sem_gb, sem_wa, sem_wb):
        wid = lax.axis_index("s") * 2 + lax.axis_index("c")
        base = wid * TPW
        pltpu.sync_copy(d0_hbm.at[wid], i0_v)
        pltpu.sync_copy(d1_hbm.at[wid], i1_v)
        pltpu.sync_copy(w0_hbm.at[pl.ds(base, TPW)], w0_v)
        pltpu.sync_copy(w1_hbm.at[pl.ds(base, TPW)], w1_v)
        r0s = (r0_a, r0_b)
        r1s = (r1_a, r1_b)
        ovs = (o_a, o_b)
        sem_g = (sem_ga, sem_gb)
        sem_w = (sem_wa, sem_wb)
        ga = {}
        wr = {}

        def start_gather(c):
            b = c % 2
            ga[c] = (
                pltpu.async_copy(ys_hbm.at[i0_v.at[c]], r0s[b], sem_g[b]),
                pltpu.async_copy(ys_hbm.at[i1_v.at[c]], r1s[b], sem_g[b]),
            )

        start_gather(0)
        for c in range(NCC):
            if c + 1 < NCC:
                if c >= 1:
                    wr.pop(c - 1).wait()
                start_gather(c + 1)
            for d in ga.pop(c):
                d.wait()
            b = c % 2
            r0_v = r0s[b]
            r1_v = r1s[b]
            o_v = ovs[b]
            w0c = w0_v[pl.ds(c * CCH, CCH)]
            w1c = w1_v[pl.ds(c * CCH, CCH)]

            @plsc.parallel_loop(0, CCH, 1)
            def _tok(t):
                w0s = _splat(w0c, t)
                w1s = _splat(w1c, t)
                for m in range(OUT_DIM // 32):
                    v0 = r0_v[t, pl.ds(m * LANES, LANES)]
                    v1 = r1_v[t, pl.ds(m * LANES, LANES)]
                    e0 = lax.bitcast_convert_type(v0 << 16, jnp.float32)
                    e1 = lax.bitcast_convert_type(v1 << 16, jnp.float32)
                    o0 = lax.bitcast_convert_type(v0 & _HIMASK, jnp.float32)
                    o1 = lax.bitcast_convert_type(v1 & _HIMASK, jnp.float32)
                    o_v[t, pl.ds(m * LANES, LANES)] = w0s * e0 + w1s * e1
                    o_v[t, pl.ds(OUT_DIM // 2 + m * LANES, LANES)] = (
                        w0s * o0 + w1s * o1)
            wr[c] = pltpu.async_copy(
                o_v, out_hbm.at[pl.ds(base + c * CCH, CCH)], sem_w[b])
        wr.pop(NCC - 2).wait()
        wr.pop(NCC - 1).wait()

    return _combine


def kernel(x, Wg, W1, b1, W2, b2):
    xb, w0, w1, d0, d1, bexp = _gate_route(x, Wg)
    d0c = jnp.reshape(d0, (NW, NCC, CCH))
    d1c = jnp.reshape(d1, (NW, NCC, CCH))
    xs = _make_dispatch()(xb, d0c, d1c)
    ys = _mlp(jnp.reshape(bexp, (NBLK,)), xs, W1, b1, W2, b2)
    out = _make_combine()(ys, d0c, d1c,
                          jnp.reshape(w0, (N_TOK,)), jnp.reshape(w1, (N_TOK,)))
    return out


# speedup vs baseline: 1.0590x; 1.0590x over previous
import functools

import jax
import jax.numpy as jnp
import numpy as np
from jax import lax
from jax.experimental import pallas as pl
from jax.experimental.pallas import tpu as pltpu
from jax.experimental.pallas import tpu_sc as plsc

N_TOK = 4096
IN_DIM = 1024
OUT_DIM = 1024
HIDDEN = 128
GATE_DIM = 4
NE = 16
BLK = 256
CAP = 12288
NBLK = CAP // BLK
GATE_T = 512
NG = N_TOK // GATE_T
NW = 32
TPW = N_TOK // NW
LANES = 16
DCH = 32
NDC = TPW // DCH
CCH = 16
NCC = TPW // CCH
_HIMASK = -65536


def _pack_bf16_pairs(lo, hi):
    bl = lax.bitcast_convert_type(lo, jnp.int32)
    bh = lax.bitcast_convert_type(hi, jnp.int32)
    rl = lax.shift_right_logical(
        bl + 0x7FFF + (lax.shift_right_logical(bl, 16) & 1), 16)
    rh = (bh + 0x7FFF + (lax.shift_right_logical(bh, 16) & 1)) & _HIMASK
    return rl | rh


def _unpack_bf16_pairs(w):
    lo = lax.bitcast_convert_type(w << 16, jnp.float32)
    hi = lax.bitcast_convert_type(w & _HIMASK, jnp.float32)
    return lo, hi


def _gate_route_body(x_ref, wg_ref, xb_ref, w0_ref, w1_ref, d0_ref, d1_ref,
                     be_ref, e0_s, e1_s):
    i = pl.program_id(0)
    x = x_ref[...]
    xb_ref[...] = _pack_bf16_pairs(x[:, :IN_DIM // 2], x[:, IN_DIM // 2:])
    logits = jnp.maximum(
        jnp.dot(x, wg_ref[...], preferred_element_type=jnp.float32), 0.0)
    g = lax.broadcasted_iota(jnp.int32, (GATE_DIM, NE), 0)
    e = lax.broadcasted_iota(jnp.int32, (GATE_DIM, NE), 1)
    mask_t = (((e >> g) & 1) * 2 - 1).astype(jnp.float32)
    s = jnp.dot(logits, mask_t, preferred_element_type=jnp.float32)
    idx = lax.broadcasted_iota(jnp.int32, s.shape, 1)
    m1 = jnp.max(s, axis=1, keepdims=True)
    i1 = jnp.min(jnp.where(s == m1, idx, NE), axis=1, keepdims=True)
    s2 = jnp.where(idx == i1, -jnp.inf, s)
    m2 = jnp.max(s2, axis=1, keepdims=True)
    i2 = jnp.min(jnp.where(s2 == m2, idx, NE), axis=1, keepdims=True)
    r = jnp.exp(m2 - m1)
    w0_ref[...] = 1.0 / (1.0 + r)
    w1_ref[...] = r / (1.0 + r)
    e0_s[pl.ds(i * GATE_T, GATE_T), :] = i1
    e1_s[pl.ds(i * GATE_T, GATE_T), :] = i2

    @pl.when(i == NG - 1)
    def _route():
        lane = lax.broadcasted_iota(jnp.int32, (N_TOK, NE), 1)
        oh0 = (e0_s[...] == lane).astype(jnp.float32)
        oh1 = (e1_s[...] == lane).astype(jnp.float32)
        c0 = jnp.sum(oh0, axis=0, keepdims=True)
        counts = c0 + jnp.sum(oh1, axis=0, keepdims=True)
        padded = jnp.ceil(counts * (1.0 / BLK)) * BLK
        r16 = lax.broadcasted_iota(jnp.int32, (NE, NE), 0)
        cidx = lax.broadcasted_iota(jnp.int32, (NE, NE), 1)
        tri = (r16 < cidx).astype(jnp.float32)
        po = jnp.dot(padded, tri, preferred_element_type=jnp.float32)
        oh = jnp.concatenate([oh0, oh1], axis=1)
        nch = 32
        chs = N_TOK // nch
        oh3 = jnp.reshape(oh, (nch, chs, 2 * NE))
        t3 = lax.broadcasted_iota(jnp.int32, (nch, chs, chs), 1)
        u3 = lax.broadcasted_iota(jnp.int32, (nch, chs, chs), 2)
        tri3 = (u3 < t3).astype(jnp.float32)
        cum3 = lax.dot_general(
            tri3, oh3, (((2,), (1,)), ((0,), (0,))),
            preferred_element_type=jnp.float32)
        tot = jnp.sum(oh3, axis=1)
        bb = lax.broadcasted_iota(jnp.int32, (nch, nch), 0)
        cc = lax.broadcasted_iota(jnp.int32, (nch, nch), 1)
        trib = (cc < bb).astype(jnp.float32)
        pre = lax.dot_general(
            trib, tot, (((1,), (0,)), ((), ())),
            preferred_element_type=jnp.float32)
        sarr = jnp.reshape(cum3 + pre[:, None, :], (N_TOK, 2 * NE))
        r0 = sarr[:, :NE]
        r1 = sarr[:, NE:]
        rank0 = jnp.sum(r0 * oh0, axis=1, keepdims=True)
        rank1 = jnp.sum((r1 + c0) * oh1, axis=1, keepdims=True)
        base0 = jnp.sum(po * oh0, axis=1, keepdims=True)
        base1 = jnp.sum(po * oh1, axis=1, keepdims=True)
        d0_ref[...] = (base0 + rank0).astype(jnp.int32)
        d1_ref[...] = (base1 + rank1).astype(jnp.int32)
        jb = (lax.broadcasted_iota(jnp.int32, (NBLK, NE), 0) * BLK).astype(
            jnp.float32)
        en = lax.broadcasted_iota(jnp.int32, (NBLK, NE), 1)
        ind = (jb >= po) & (jb < po + padded)
        be_ref[...] = jnp.sum(jnp.where(ind, en, 0), axis=1, keepdims=True)


def _gate_route(x, wg, interpret=False):
    return pl.pallas_call(
        _gate_route_body,
        grid=(NG,),
        in_specs=[
            pl.BlockSpec((GATE_T, IN_DIM), lambda i: (i, 0)),
            pl.BlockSpec((IN_DIM, GATE_DIM), lambda i: (0, 0)),
        ],
        out_specs=[
            pl.BlockSpec((GATE_T, IN_DIM // 2), lambda i: (i, 0)),
            pl.BlockSpec((GATE_T, 1), lambda i: (i, 0)),
            pl.BlockSpec((GATE_T, 1), lambda i: (i, 0)),
            pl.BlockSpec((N_TOK, 1), lambda i: (0, 0)),
            pl.BlockSpec((N_TOK, 1), lambda i: (0, 0)),
            pl.BlockSpec((NBLK, 1), lambda i: (0, 0)),
        ],
        out_shape=[
            jax.ShapeDtypeStruct((N_TOK, IN_DIM // 2), jnp.int32),
            jax.ShapeDtypeStruct((N_TOK, 1), jnp.float32),
            jax.ShapeDtypeStruct((N_TOK, 1), jnp.float32),
            jax.ShapeDtypeStruct((N_TOK, 1), jnp.int32),
            jax.ShapeDtypeStruct((N_TOK, 1), jnp.int32),
            jax.ShapeDtypeStruct((NBLK, 1), jnp.int32),
        ],
        scratch_shapes=[
            pltpu.VMEM((N_TOK, 1), jnp.int32),
            pltpu.VMEM((N_TOK, 1), jnp.int32),
        ],
        interpret=interpret,
    )(x, wg)


@functools.cache
def _make_dispatch():
    mesh = plsc.VectorSubcoreMesh(core_axis_name="c", subcore_axis_name="s")

    @functools.partial(
        pl.kernel,
        out_type=jax.ShapeDtypeStruct((CAP, IN_DIM // 2), jnp.int32),
        mesh=mesh,
        scratch_types=[
            pltpu.VMEM((NDC, DCH), jnp.int32),
            pltpu.VMEM((NDC, DCH), jnp.int32),
            pltpu.VMEM((DCH, IN_DIM // 2), jnp.int32),
            pltpu.VMEM((DCH, IN_DIM // 2), jnp.int32),
            pltpu.SemaphoreType.DMA,
            pltpu.SemaphoreType.DMA,
            pltpu.SemaphoreType.DMA,
            pltpu.SemaphoreType.DMA,
        ],
    )
    def _dispatch(x_hbm, d0_hbm, d1_hbm, xs_hbm, i0_v, i1_v, rows_a, rows_b,
                  sem_ra, sem_rb, sem_sa, sem_sb):
        wid = lax.axis_index("s") * 2 + lax.axis_index("c")
        base = wid * TPW
        pltpu.sync_copy(d0_hbm.at[wid], i0_v)
        pltpu.sync_copy(d1_hbm.at[wid], i1_v)
        rows = (rows_a, rows_b)
        sem_r = (sem_ra, sem_rb)
        sem_s = (sem_sa, sem_sb)
        rd = {}
        sc = {}
        rd[0] = pltpu.async_copy(
            x_hbm.at[pl.ds(base, DCH)], rows[0], sem_r[0])
        for c in range(NDC):
            if c >= 1:
                for d in sc.pop(c - 1):
                    d.wait()
            if c + 1 < NDC:
                b = (c + 1) % 2
                rd[c + 1] = pltpu.async_copy(
                    x_hbm.at[pl.ds(base + (c + 1) * DCH, DCH)],
                    rows[b], sem_r[b])
            rd.pop(c).wait()
            b = c % 2
            sc[c] = (
                pltpu.async_copy(rows[b], xs_hbm.at[i0_v.at[c]], sem_s[b]),
                pltpu.async_copy(rows[b], xs_hbm.at[i1_v.at[c]], sem_s[b]),
            )
        for d in sc.pop(NDC - 1):
            d.wait()

    return _dispatch


def _mlp_body(be_ref, xs_ref, w1_ref, b1_ref, w2_ref, b2_ref, ys_ref):
    del be_ref
    lo, hi = _unpack_bf16_pairs(xs_ref[...])
    xb = jnp.concatenate(
        [lo.astype(jnp.bfloat16), hi.astype(jnp.bfloat16)], axis=1)
    h = jnp.maximum(
        jnp.dot(xb, w1_ref[0].astype(jnp.bfloat16),
                preferred_element_type=jnp.float32) + b1_ref[0], 0.0)
    ys = jnp.dot(h.astype(jnp.bfloat16), w2_ref[0].astype(jnp.bfloat16),
                 preferred_element_type=jnp.float32) + b2_ref[0]
    ys_ref[...] = _pack_bf16_pairs(ys[:, :OUT_DIM // 2], ys[:, OUT_DIM // 2:])


def _mlp(bexp, xs, w1b, b1, w2b, b2, interpret=False):
    grid_spec = pltpu.PrefetchScalarGridSpec(
        num_scalar_prefetch=1,
        grid=(NBLK,),
        in_specs=[
            pl.BlockSpec((BLK, IN_DIM // 2), lambda i, be: (i, 0)),
            pl.BlockSpec((1, IN_DIM, HIDDEN), lambda i, be: (be[i], 0, 0)),
            pl.BlockSpec((1, 1, HIDDEN), lambda i, be: (be[i], 0, 0)),
            pl.BlockSpec((1, HIDDEN, OUT_DIM), lambda i, be: (be[i], 0, 0)),
            pl.BlockSpec((1, 1, OUT_DIM), lambda i, be: (be[i], 0, 0)),
        ],
        out_specs=pl.BlockSpec((BLK, OUT_DIM // 2), lambda i, be: (i, 0)),
    )
    return pl.pallas_call(
        _mlp_body,
        grid_spec=grid_spec,
        out_shape=jax.ShapeDtypeStruct((CAP, OUT_DIM // 2), jnp.int32),
        interpret=interpret,
    )(bexp, xs, w1b, jnp.reshape(b1, (NE, 1, HIDDEN)),
      w2b, jnp.reshape(b2, (NE, 1, OUT_DIM)))


def _splat(vec, i):
    dnums = lax.GatherDimensionNumbers(
        offset_dims=(), collapsed_slice_dims=(0,), start_index_map=(0,))
    idx = jnp.full((LANES, 1), i, jnp.int32)
    return lax.gather(vec, idx, dnums, slice_sizes=(1,),
                      mode=lax.GatherScatterMode.PROMISE_IN_BOUNDS)


@functools.cache
def _make_combine():
    mesh = plsc.VectorSubcoreMesh(core_axis_name="c", subcore_axis_name="s")

    @functools.partial(
        pl.kernel,
        out_type=jax.ShapeDtypeStruct((N_TOK, OUT_DIM), jnp.float32),
        mesh=mesh,
        scratch_types=[
            pltpu.VMEM((NCC, CCH), jnp.int32),
            pltpu.VMEM((NCC, CCH), jnp.int32),
            pltpu.VMEM((TPW,), jnp.float32),
            pltpu.VMEM((TPW,), jnp.float32),
            pltpu.VMEM((CCH, OUT_DIM // 2), jnp.int32),
            pltpu.VMEM((CCH, OUT_DIM // 2), jnp.int32),
            pltpu.VMEM((CCH, OUT_DIM // 2), jnp.int32),
            pltpu.VMEM((CCH, OUT_DIM // 2), jnp.int32),
            pltpu.VMEM((CCH, OUT_DIM), jnp.float32),
            pltpu.VMEM((CCH, OUT_DIM), jnp.float32),
            pltpu.SemaphoreType.DMA,
            pltpu.SemaphoreType.DMA,
            pltpu.SemaphoreType.DMA,
            pltpu.SemaphoreType.DMA,
        ],
    )
    def _combine(ys_hbm, d0_hbm, d1_hbm, w0_hbm, w1_hbm, out_hbm,
                 i0_v, i1_v, w0_v, w1_v, r0_a, r1_a, r0_b, r1_b, o_a, o_b,
                 sem_ga, sem_gb, sem_wa, sem_wb):
        wid = lax.axis_index("s") * 2 + lax.axis_index("c")
        base = wid * TPW
        pltpu.sync_copy(d0_hbm.at[wid], i0_v)
        pltpu.sync_copy(d1_hbm.at[wid], i1_v)
        pltpu.sync_copy(w0_hbm.at[pl.ds(base, TPW)], w0_v)
        pltpu.sync_copy(w1_hbm.at[pl.ds(base, TPW)], w1_v)
        r0s = (r0_a, r0_b)
        r1s = (r1_a, r1_b)
        ovs = (o_a, o_b)
        sem_g = (sem_ga, sem_gb)
        sem_w = (sem_wa, sem_wb)
        ga = {}
        wr = {}

        def start_gather(c):
            b = c % 2
            ga[c] = (
                pltpu.async_copy(ys_hbm.at[i0_v.at[c]], r0s[b], sem_g[b]),
                pltpu.async_copy(ys_hbm.at[i1_v.at[c]], r1s[b], sem_g[b]),
            )

        start_gather(0)
        for c in range(NCC):
            if c + 1 < NCC:
                if c >= 1:
                    wr.pop(c - 1).wait()
                start_gather(c + 1)
            for d in ga.pop(c):
                d.wait()
            b = c % 2
            r0_v = r0s[b]
            r1_v = r1s[b]
            o_v = ovs[b]
            w0c = w0_v[pl.ds(c * CCH, CCH)]
            w1c = w1_v[pl.ds(c * CCH, CCH)]

            @plsc.parallel_loop(0, CCH, 1)
            def _tok(t):
                w0s = _splat(w0c, t)
                w1s = _splat(w1c, t)
                for m in range(OUT_DIM // 32):
                    v0 = r0_v[t, pl.ds(m * LANES, LANES)]
                    v1 = r1_v[t, pl.ds(m * LANES, LANES)]
                    e0 = lax.bitcast_convert_type(v0 << 16, jnp.float32)
                    e1 = lax.bitcast_convert_type(v1 << 16, jnp.float32)
                    o0 = lax.bitcast_convert_type(v0 & _HIMASK, jnp.float32)
                    o1 = lax.bitcast_convert_type(v1 & _HIMASK, jnp.float32)
                    o_v[t, pl.ds(m * LANES, LANES)] = w0s * e0 + w1s * e1
                    o_v[t, pl.ds(OUT_DIM // 2 + m * LANES, LANES)] = (
                        w0s * o0 + w1s * o1)
            wr[c] = pltpu.async_copy(
                o_v, out_hbm.at[pl.ds(base + c * CCH, CCH)], sem_w[b])
        wr.pop(NCC - 2).wait()
        wr.pop(NCC - 1).wait()

    return _combine


def kernel(x, Wg, W1, b1, W2, b2):
    xb, w0, w1, d0, d1, bexp = _gate_route(x, Wg)
    d0d = jnp.reshape(d0, (NW, NDC, DCH))
    d1d = jnp.reshape(d1, (NW, NDC, DCH))
    xs = _make_dispatch()(xb, d0d, d1d)
    ys = _mlp(jnp.reshape(bexp, (NBLK,)), xs, W1, b1, W2, b2)
    d0c = jnp.reshape(d0, (NW, NCC, CCH))
    d1c = jnp.reshape(d1, (NW, NCC, CCH))
    out = _make_combine()(ys, d0c, d1c,
                          jnp.reshape(w0, (N_TOK,)), jnp.reshape(w1, (N_TOK,)))
    return out
